# Initial kernel scaffold; baseline (speedup 1.0000x reference)
import jax
import jax.numpy as jnp
from jax.experimental import pallas as pl


def kernel(input_token, position_ids, segment_ids, tok_table, type_table, pos_table, ln_gamma, ln_beta):
    raise NotImplementedError("write your pallas kernel here")


# speedup vs baseline: 8.2589x; 8.2589x over previous
import functools

import jax
import jax.numpy as jnp
from jax import lax
from jax.experimental import pallas as pl
from jax.experimental.pallas import tpu as pltpu
from jax.experimental.pallas import tpu_sc as plsc

B = 1024
L = 200
D = 64
ROWS = B * L
GRP = 128
NG = ROWS // GRP
NC = 2
NS = 16
NW = NC * NS
GPW = NG // NW
EPS = 1e-5
LN_ROWS = 2048


def _ptab_body(pos_ref, typ_ref, out_ref):
    p = pos_ref[...]
    out_ref[0] = p + typ_ref[0:1, :]
    out_ref[1] = p + typ_ref[1:2, :]


def _build_ptab(pos_table, type_table):
    out = pl.pallas_call(
        _ptab_body,
        out_shape=jax.ShapeDtypeStruct((2, L, D), jnp.float32),
    )(pos_table, type_table)
    return out.reshape(2 * L, D)


def _sc_body(tok_hbm, ptab_hbm, tidx_hbm, pidx_hbm, sidx_hbm, out_hbm,
             tidx_v, pidx_v, sidx_v, cidx_v, trow_v, prow_v, sem_t, sem_p):
    wid = lax.axis_index("s") * NC + lax.axis_index("c")
    g0 = wid * GPW

    pltpu.sync_copy(tidx_hbm.at[wid], tidx_v)
    pltpu.sync_copy(pidx_hbm.at[wid], pidx_v)
    pltpu.sync_copy(sidx_hbm.at[wid], sidx_v)

    def _cidx_body(g, _):
        for j in range(GRP // 16):
            sl = pl.ds(j * 16, 16)
            cidx_v[g, sl] = sidx_v[g, sl] * L + pidx_v[g, sl]
        return 0

    lax.fori_loop(0, GPW, _cidx_body, 0)

    def _group_body(g, _):
        cp_t = pltpu.async_copy(tok_hbm.at[tidx_v.at[g]], trow_v, sem_t)
        cp_p = pltpu.async_copy(ptab_hbm.at[cidx_v.at[g]], prow_v, sem_p)
        cp_t.wait()
        cp_p.wait()

        def _row_body(r, _):
            for q in range(D // 16):
                sl = pl.ds(q * 16, 16)
                trow_v[r, sl] = trow_v[r, sl] + prow_v[r, sl]
            return 0

        lax.fori_loop(0, GRP, _row_body, 0)
        pltpu.sync_copy(trow_v, out_hbm.at[pl.ds((g0 + g) * GRP, GRP)])
        return 0

    lax.fori_loop(0, GPW, _group_body, 0)


def _sc_gather_sum(tok_table, ptab, tidx, pidx, sidx):
    mesh = plsc.VectorSubcoreMesh(core_axis_name="c", subcore_axis_name="s")
    fn = functools.partial(
        pl.kernel,
        mesh=mesh,
        compiler_params=pltpu.CompilerParams(use_tc_tiling_on_sc=False),
        out_type=jax.ShapeDtypeStruct((ROWS, D), jnp.float32),
        scratch_types=[
            pltpu.VMEM((GPW, GRP), jnp.int32),
            pltpu.VMEM((GPW, GRP), jnp.int32),
            pltpu.VMEM((GPW, GRP), jnp.int32),
            pltpu.VMEM((GPW, GRP), jnp.int32),
            pltpu.VMEM((GRP, D), jnp.float32),
            pltpu.VMEM((GRP, D), jnp.float32),
            pltpu.SemaphoreType.DMA,
            pltpu.SemaphoreType.DMA,
        ],
    )(_sc_body)
    return fn(tok_table, ptab, tidx, pidx, sidx)


def _ln_body(x_ref, g_ref, b_ref, m_ref, y_ref, mask_ref):
    x = x_ref[...]
    mean = jnp.mean(x, axis=1, keepdims=True)
    xc = x - mean
    var = jnp.mean(xc * xc, axis=1, keepdims=True)
    inv = lax.rsqrt(var + EPS)
    y_ref[...] = xc * inv * g_ref[...] + b_ref[...]
    mask_ref[...] = jnp.broadcast_to(m_ref[...], x.shape)


def _ln_and_mask(summed, gamma, beta, mask_row):
    grid = (ROWS // LN_ROWS,)
    return pl.pallas_call(
        _ln_body,
        grid=grid,
        in_specs=[
            pl.BlockSpec((LN_ROWS, D), lambda i: (i, 0)),
            pl.BlockSpec((1, D), lambda i: (0, 0)),
            pl.BlockSpec((1, D), lambda i: (0, 0)),
            pl.BlockSpec((1, D), lambda i: (0, 0)),
        ],
        out_specs=[
            pl.BlockSpec((LN_ROWS, D), lambda i: (i, 0)),
            pl.BlockSpec((LN_ROWS, D), lambda i: (i, 0)),
        ],
        out_shape=[
            jax.ShapeDtypeStruct((ROWS, D), jnp.float32),
            jax.ShapeDtypeStruct((ROWS, D), jnp.float32),
        ],
    )(summed, gamma, beta, mask_row)


def kernel(input_token, position_ids, segment_ids, tok_table, type_table,
           pos_table, ln_gamma, ln_beta):
    tidx = input_token.reshape(NW, GPW, GRP)
    pidx = position_ids.reshape(NW, GPW, GRP)
    sidx = segment_ids.reshape(NW, GPW, GRP)

    ptab = _build_ptab(pos_table, type_table)
    summed = _sc_gather_sum(tok_table, ptab, tidx, pidx, sidx)

    mask_row = lax.slice(tok_table, (103, 0), (104, D))
    y, mask = _ln_and_mask(summed, ln_gamma.reshape(1, D),
                           ln_beta.reshape(1, D), mask_row)
    return y.reshape(B, L, D), mask.reshape(B, L, D)
